# baseline (speedup 1.0000x reference)
import functools

import jax
import jax.numpy as jnp
from jax import lax
from jax.experimental import pallas as pl
from jax.experimental.pallas import tpu as pltpu
from jax.experimental.pallas import tpu_sc as plsc

N = 10000
E = 160000
D = 256
DH = 128
NC = 2
NS = 16
CH = 128
AGG_CH = 80
SS = 40
DEG_CH = 40
E_PAD = NC * NS * DEG_CH * CH
NP = 10112
ROWS_PER_TILE = NP // NS


def _deg_body(dst_hbm, zeros_hbm, ones_hbm, degp_hbm, dstv, onesv, acc):
    c = lax.axis_index("c")
    s = lax.axis_index("s")
    wid = s * NC + c
    pltpu.sync_copy(dst_hbm.at[wid], dstv)
    pltpu.sync_copy(ones_hbm, onesv)

    @pl.when(s == 0)
    def _():
        pltpu.sync_copy(zeros_hbm, acc)

    plsc.subcore_barrier()

    def step(j, carry):
        pltpu.sync_copy(onesv, acc.at[dstv.at[j]], add=True)
        return carry

    lax.fori_loop(0, DEG_CH, step, 0)
    plsc.subcore_barrier()

    @pl.when(s == 0)
    def _():
        pltpu.sync_copy(acc, degp_hbm.at[c])


@functools.cache
def _sc_mesh():
    return plsc.VectorSubcoreMesh(
        core_axis_name="c", subcore_axis_name="s", num_cores=NC, num_subcores=NS
    )


@functools.cache
def _deg_call():
    return pl.kernel(
        _deg_body,
        out_type=jax.ShapeDtypeStruct((NC, NP), jnp.float32),
        mesh=_sc_mesh(),
        scratch_types=[
            pltpu.VMEM((DEG_CH, CH), jnp.int32),
            pltpu.VMEM((CH,), jnp.float32),
            pltpu.VMEM_SHARED((NP,), jnp.float32),
        ],
    )


def _agg_body(y0_hbm, y1_hbm, src_hbm, dst_hbm, zeros_hbm, agg_hbm,
              srcv, dstv, gbuf0, gbuf1, acc, gsem0, gsem1):
    c = lax.axis_index("c")
    s = lax.axis_index("s")

    @pl.when(s == 0)
    def _():
        pltpu.sync_copy(zeros_hbm, acc)

    plsc.subcore_barrier()

    def run(y_hbm):
        def superstep(q, carry):
            pltpu.sync_copy(src_hbm.at[s].at[pl.ds(q * SS, SS)], srcv)
            pltpu.sync_copy(dst_hbm.at[s].at[pl.ds(q * SS, SS)], dstv)

            def gather2(j, gbuf, gsem):
                pltpu.async_copy(y_hbm.at[srcv.at[j, pl.ds(0, CH // 2)]],
                                 gbuf.at[pl.ds(0, CH // 2)], gsem)
                pltpu.async_copy(y_hbm.at[srcv.at[j, pl.ds(CH // 2, CH // 2)]],
                                 gbuf.at[pl.ds(CH // 2, CH // 2)], gsem)

            def wait2(j, gbuf, gsem):
                pltpu.make_async_copy(y_hbm.at[srcv.at[j, pl.ds(0, CH // 2)]],
                                      gbuf.at[pl.ds(0, CH // 2)], gsem).wait()
                pltpu.make_async_copy(y_hbm.at[srcv.at[j, pl.ds(0, CH // 2)]],
                                      gbuf.at[pl.ds(0, CH // 2)], gsem).wait()

            gather2(0, gbuf0, gsem0)
            gather2(1, gbuf1, gsem1)

            def pair(k, carry2):
                wait2(2 * k, gbuf0, gsem0)
                pltpu.sync_copy(gbuf0, acc.at[dstv.at[2 * k]], add=True)

                @pl.when(k < SS // 2 - 1)
                def _():
                    gather2(2 * k + 2, gbuf0, gsem0)

                wait2(2 * k + 1, gbuf1, gsem1)
                pltpu.sync_copy(gbuf1, acc.at[dstv.at[2 * k + 1]], add=True)

                @pl.when(k < SS // 2 - 1)
                def _():
                    gather2(2 * k + 3, gbuf1, gsem1)

                return carry2

            lax.fori_loop(0, SS // 2, pair, 0)
            return carry

        lax.fori_loop(0, AGG_CH // SS, superstep, 0)

    @pl.when(c == 0)
    def _():
        run(y0_hbm)

    @pl.when(c == 1)
    def _():
        run(y1_hbm)

    plsc.subcore_barrier()

    @pl.when(c == 0)
    def _():
        pltpu.sync_copy(acc.at[pl.ds(s * ROWS_PER_TILE, ROWS_PER_TILE)],
                        agg_hbm.at[0].at[pl.ds(s * ROWS_PER_TILE, ROWS_PER_TILE)])

    @pl.when(c == 1)
    def _():
        pltpu.sync_copy(acc.at[pl.ds(s * ROWS_PER_TILE, ROWS_PER_TILE)],
                        agg_hbm.at[1].at[pl.ds(s * ROWS_PER_TILE, ROWS_PER_TILE)])


@functools.cache
def _agg_call():
    return pl.kernel(
        _agg_body,
        out_type=jax.ShapeDtypeStruct((NC, NP, DH), jnp.float32),
        mesh=_sc_mesh(),
        scratch_types=[
            pltpu.VMEM((SS, CH), jnp.int32),
            pltpu.VMEM((SS, CH), jnp.int32),
            pltpu.VMEM((CH, DH), jnp.float32),
            pltpu.VMEM((CH, DH), jnp.float32),
            pltpu.VMEM_SHARED((NP, DH), jnp.float32),
            pltpu.SemaphoreType.DMA,
            pltpu.SemaphoreType.DMA,
        ],
    )


BR = 2000


def _tc1_body(x_ref, w_ref, b_ref, degp_ref, y0_ref, y1_ref, z_ref):
    deg = degp_ref[:, 0:1] + degp_ref[:, 1:2] + 1.0
    dis = lax.rsqrt(deg)
    inv = 1.0 / deg
    xw = jnp.dot(x_ref[...], w_ref[...], preferred_element_type=jnp.float32)
    y = xw * dis
    y0_ref[...] = y[:, :DH]
    y1_ref[...] = y[:, DH:]
    z_ref[...] = xw * inv + b_ref[...]


def _tc2_body(agg0_ref, agg1_ref, z1_ref, degp_ref, w_ref, b_ref,
              y0_ref, y1_ref, z2_ref):
    deg = degp_ref[:, 0:1] + degp_ref[:, 1:2] + 1.0
    dis = lax.rsqrt(deg)
    inv = 1.0 / deg
    agg = jnp.concatenate([agg0_ref[0], agg1_ref[0]], axis=1)
    h = jnp.maximum(agg * dis + z1_ref[...], 0.0)
    xw = jnp.dot(h, w_ref[...], preferred_element_type=jnp.float32)
    y = xw * dis
    y0_ref[...] = y[:, :DH]
    y1_ref[...] = y[:, DH:]
    z2_ref[...] = xw * inv + b_ref[...]


def _tc3_body(agg0_ref, agg1_ref, z2_ref, degp_ref, o_ref):
    deg = degp_ref[:, 0:1] + degp_ref[:, 1:2] + 1.0
    dis = lax.rsqrt(deg)
    agg = jnp.concatenate([agg0_ref[0], agg1_ref[0]], axis=1)
    o_ref[...] = jnp.maximum(agg * dis + z2_ref[...], 0.0)


def _row_spec(cols):
    return pl.BlockSpec((BR, cols), lambda i: (i, 0))


def _full_spec(shape):
    return pl.BlockSpec(shape, lambda i: tuple(0 for _ in shape))


_GRID = (N // BR,)

_tc1_call = pl.pallas_call(
    _tc1_body,
    grid=_GRID,
    in_specs=[_row_spec(D), _full_spec((D, D)), _full_spec((1, D)), _row_spec(2)],
    out_specs=[_row_spec(DH), _row_spec(DH), _row_spec(D)],
    out_shape=[
        jax.ShapeDtypeStruct((N, DH), jnp.float32),
        jax.ShapeDtypeStruct((N, DH), jnp.float32),
        jax.ShapeDtypeStruct((N, D), jnp.float32),
    ],
)

def _agg_spec(core):
    return pl.BlockSpec((1, BR, DH), lambda i, _c=core: (_c, i, 0))


_tc2_call = pl.pallas_call(
    _tc2_body,
    grid=_GRID,
    in_specs=[_agg_spec(0), _agg_spec(1), _row_spec(D), _row_spec(2),
              _full_spec((D, D)), _full_spec((1, D))],
    out_specs=[_row_spec(DH), _row_spec(DH), _row_spec(D)],
    out_shape=[
        jax.ShapeDtypeStruct((N, DH), jnp.float32),
        jax.ShapeDtypeStruct((N, DH), jnp.float32),
        jax.ShapeDtypeStruct((N, D), jnp.float32),
    ],
)

_tc3_call = pl.pallas_call(
    _tc3_body,
    grid=_GRID,
    in_specs=[_agg_spec(0), _agg_spec(1), _row_spec(D), _row_spec(2)],
    out_specs=_row_spec(D),
    out_shape=jax.ShapeDtypeStruct((N, D), jnp.float32),
)


def kernel(x, adj, W1, b1, W2, b2):
    adj = adj.astype(jnp.int32)
    src = adj[0]
    dst = adj[1]
    pad = E_PAD - E
    srcp = jnp.concatenate([src, jnp.zeros((pad,), jnp.int32)])
    dstp = jnp.concatenate([dst, jnp.full((pad,), N, jnp.int32)])
    src_a = srcp.reshape(NS, AGG_CH, CH)
    dst_a = dstp.reshape(NS, AGG_CH, CH)
    dst_d = dstp.reshape(NC * NS, DEG_CH, CH)

    zeros1 = jnp.zeros((NP,), jnp.float32)
    zeros2 = jnp.zeros((NP, DH), jnp.float32)
    ones = jnp.ones((CH,), jnp.float32)
    b1r = b1.reshape(1, D)
    b2r = b2.reshape(1, D)

    degp = _deg_call()(dst_d, zeros1, ones)
    degp_t = jnp.transpose(degp[:, :N])

    y1_0, y1_1, z1 = _tc1_call(x, W1, b1r, degp_t)
    agg1 = _agg_call()(y1_0, y1_1, src_a, dst_a, zeros2)
    y2_0, y2_1, z2 = _tc2_call(agg1, agg1, z1, degp_t, W2, b2r)
    agg2 = _agg_call()(y2_0, y2_1, src_a, dst_a, zeros2)
    out = _tc3_call(agg2, agg2, z2, degp_t)
    return out


# speedup vs baseline: 1.0003x; 1.0003x over previous
import functools

import jax
import jax.numpy as jnp
from jax import lax
from jax.experimental import pallas as pl
from jax.experimental.pallas import tpu as pltpu
from jax.experimental.pallas import tpu_sc as plsc

N = 10000
E = 160000
D = 256
DH = 128
NC = 2
NS = 16
CH = 128
AGG_CH = 80
SS = 40
DEG_CH = 40
E_PAD = NC * NS * DEG_CH * CH
NP = 10112
ROWS_PER_TILE = NP // NS


def _deg_body(dst_hbm, zeros_hbm, ones_hbm, degp_hbm, dstv, onesv, acc):
    c = lax.axis_index("c")
    s = lax.axis_index("s")
    wid = s * NC + c
    pltpu.sync_copy(dst_hbm.at[wid], dstv)
    pltpu.sync_copy(ones_hbm, onesv)

    @pl.when(s == 0)
    def _():
        pltpu.sync_copy(zeros_hbm, acc)

    plsc.subcore_barrier()

    def step(j, carry):
        pltpu.sync_copy(onesv, acc.at[dstv.at[j]], add=True)
        return carry

    lax.fori_loop(0, DEG_CH, step, 0)
    plsc.subcore_barrier()

    @pl.when(s == 0)
    def _():
        pltpu.sync_copy(acc, degp_hbm.at[c])


@functools.cache
def _sc_mesh():
    return plsc.VectorSubcoreMesh(
        core_axis_name="c", subcore_axis_name="s", num_cores=NC, num_subcores=NS
    )


@functools.cache
def _deg_call():
    return pl.kernel(
        _deg_body,
        out_type=jax.ShapeDtypeStruct((NC, NP), jnp.float32),
        mesh=_sc_mesh(),
        scratch_types=[
            pltpu.VMEM((DEG_CH, CH), jnp.int32),
            pltpu.VMEM((CH,), jnp.float32),
            pltpu.VMEM_SHARED((NP,), jnp.float32),
        ],
    )


def _agg_body(y0_hbm, y1_hbm, src_hbm, dst_hbm, zeros_hbm, agg_hbm,
              srcv, dstv, gbuf0, gbuf1, acc, gsem0, gsem1):
    c = lax.axis_index("c")
    s = lax.axis_index("s")
    pltpu.sync_copy(zeros_hbm.at[pl.ds(s * ROWS_PER_TILE, ROWS_PER_TILE)],
                    acc.at[pl.ds(s * ROWS_PER_TILE, ROWS_PER_TILE)])
    plsc.subcore_barrier()

    def run(y_hbm):
        def superstep(q, carry):
            pltpu.sync_copy(src_hbm.at[s].at[pl.ds(q * SS, SS)], srcv)
            pltpu.sync_copy(dst_hbm.at[s].at[pl.ds(q * SS, SS)], dstv)
            pltpu.async_copy(y_hbm.at[srcv.at[0]], gbuf0, gsem0)
            pltpu.async_copy(y_hbm.at[srcv.at[1]], gbuf1, gsem1)

            def pair(k, carry2):
                pltpu.make_async_copy(y_hbm.at[srcv.at[2 * k]], gbuf0, gsem0).wait()
                pltpu.sync_copy(gbuf0, acc.at[dstv.at[2 * k]], add=True)

                @pl.when(k < SS // 2 - 1)
                def _():
                    pltpu.async_copy(y_hbm.at[srcv.at[2 * k + 2]], gbuf0, gsem0)

                pltpu.make_async_copy(y_hbm.at[srcv.at[2 * k + 1]], gbuf1, gsem1).wait()
                pltpu.sync_copy(gbuf1, acc.at[dstv.at[2 * k + 1]], add=True)

                @pl.when(k < SS // 2 - 1)
                def _():
                    pltpu.async_copy(y_hbm.at[srcv.at[2 * k + 3]], gbuf1, gsem1)

                return carry2

            lax.fori_loop(0, SS // 2, pair, 0)
            return carry

        lax.fori_loop(0, AGG_CH // SS, superstep, 0)

    @pl.when(c == 0)
    def _():
        run(y0_hbm)

    @pl.when(c == 1)
    def _():
        run(y1_hbm)

    plsc.subcore_barrier()

    @pl.when(c == 0)
    def _():
        pltpu.sync_copy(acc.at[pl.ds(s * ROWS_PER_TILE, ROWS_PER_TILE)],
                        agg_hbm.at[0].at[pl.ds(s * ROWS_PER_TILE, ROWS_PER_TILE)])

    @pl.when(c == 1)
    def _():
        pltpu.sync_copy(acc.at[pl.ds(s * ROWS_PER_TILE, ROWS_PER_TILE)],
                        agg_hbm.at[1].at[pl.ds(s * ROWS_PER_TILE, ROWS_PER_TILE)])


@functools.cache
def _agg_call():
    return pl.kernel(
        _agg_body,
        out_type=jax.ShapeDtypeStruct((NC, NP, DH), jnp.float32),
        mesh=_sc_mesh(),
        scratch_types=[
            pltpu.VMEM((SS, CH), jnp.int32),
            pltpu.VMEM((SS, CH), jnp.int32),
            pltpu.VMEM((CH, DH), jnp.float32),
            pltpu.VMEM((CH, DH), jnp.float32),
            pltpu.VMEM_SHARED((NP, DH), jnp.float32),
            pltpu.SemaphoreType.DMA,
            pltpu.SemaphoreType.DMA,
        ],
    )


BR = 2000


def _tc1_body(x_ref, w_ref, b_ref, degp_ref, y0_ref, y1_ref, z_ref):
    deg = degp_ref[:, 0:1] + degp_ref[:, 1:2] + 1.0
    dis = lax.rsqrt(deg)
    inv = 1.0 / deg
    xw = jnp.dot(x_ref[...], w_ref[...], preferred_element_type=jnp.float32)
    y = xw * dis
    y0_ref[...] = y[:, :DH]
    y1_ref[...] = y[:, DH:]
    z_ref[...] = xw * inv + b_ref[...]


def _tc2_body(agg0_ref, agg1_ref, z1_ref, degp_ref, w_ref, b_ref,
              y0_ref, y1_ref, z2_ref):
    deg = degp_ref[:, 0:1] + degp_ref[:, 1:2] + 1.0
    dis = lax.rsqrt(deg)
    inv = 1.0 / deg
    agg = jnp.concatenate([agg0_ref[0], agg1_ref[0]], axis=1)
    h = jnp.maximum(agg * dis + z1_ref[...], 0.0)
    xw = jnp.dot(h, w_ref[...], preferred_element_type=jnp.float32)
    y = xw * dis
    y0_ref[...] = y[:, :DH]
    y1_ref[...] = y[:, DH:]
    z2_ref[...] = xw * inv + b_ref[...]


def _tc3_body(agg0_ref, agg1_ref, z2_ref, degp_ref, o_ref):
    deg = degp_ref[:, 0:1] + degp_ref[:, 1:2] + 1.0
    dis = lax.rsqrt(deg)
    agg = jnp.concatenate([agg0_ref[0], agg1_ref[0]], axis=1)
    o_ref[...] = jnp.maximum(agg * dis + z2_ref[...], 0.0)


def _row_spec(cols):
    return pl.BlockSpec((BR, cols), lambda i: (i, 0))


def _full_spec(shape):
    return pl.BlockSpec(shape, lambda i: tuple(0 for _ in shape))


_GRID = (N // BR,)

_tc1_call = pl.pallas_call(
    _tc1_body,
    grid=_GRID,
    in_specs=[_row_spec(D), _full_spec((D, D)), _full_spec((1, D)), _row_spec(2)],
    out_specs=[_row_spec(DH), _row_spec(DH), _row_spec(D)],
    out_shape=[
        jax.ShapeDtypeStruct((N, DH), jnp.float32),
        jax.ShapeDtypeStruct((N, DH), jnp.float32),
        jax.ShapeDtypeStruct((N, D), jnp.float32),
    ],
)

def _agg_spec(core):
    return pl.BlockSpec((1, BR, DH), lambda i, _c=core: (_c, i, 0))


_tc2_call = pl.pallas_call(
    _tc2_body,
    grid=_GRID,
    in_specs=[_agg_spec(0), _agg_spec(1), _row_spec(D), _row_spec(2),
              _full_spec((D, D)), _full_spec((1, D))],
    out_specs=[_row_spec(DH), _row_spec(DH), _row_spec(D)],
    out_shape=[
        jax.ShapeDtypeStruct((N, DH), jnp.float32),
        jax.ShapeDtypeStruct((N, DH), jnp.float32),
        jax.ShapeDtypeStruct((N, D), jnp.float32),
    ],
)

_tc3_call = pl.pallas_call(
    _tc3_body,
    grid=_GRID,
    in_specs=[_agg_spec(0), _agg_spec(1), _row_spec(D), _row_spec(2)],
    out_specs=_row_spec(D),
    out_shape=jax.ShapeDtypeStruct((N, D), jnp.float32),
)


def kernel(x, adj, W1, b1, W2, b2):
    adj = adj.astype(jnp.int32)
    src = adj[0]
    dst = adj[1]
    pad = E_PAD - E
    srcp = jnp.concatenate([src, jnp.zeros((pad,), jnp.int32)])
    dstp = jnp.concatenate([dst, jnp.full((pad,), N, jnp.int32)])
    src_a = srcp.reshape(NS, AGG_CH, CH)
    dst_a = dstp.reshape(NS, AGG_CH, CH)
    dst_d = dstp.reshape(NC * NS, DEG_CH, CH)

    zeros1 = jnp.zeros((NP,), jnp.float32)
    zeros2 = jnp.zeros((NP, DH), jnp.float32)
    ones = jnp.ones((CH,), jnp.float32)
    b1r = b1.reshape(1, D)
    b2r = b2.reshape(1, D)

    degp = _deg_call()(dst_d, zeros1, ones)
    degp_t = jnp.transpose(degp[:, :N])

    y1_0, y1_1, z1 = _tc1_call(x, W1, b1r, degp_t)
    agg1 = _agg_call()(y1_0, y1_1, src_a, dst_a, zeros2)
    y2_0, y2_1, z2 = _tc2_call(agg1, agg1, z1, degp_t, W2, b2r)
    agg2 = _agg_call()(y2_0, y2_1, src_a, dst_a, zeros2)
    out = _tc3_call(agg2, agg2, z2, degp_t)
    return out
